# baseline (device time: 866618 ns/iter reference)
import os

import jax
import jax.numpy as jnp
from jax import lax
from jax.experimental import pallas as pl
from jax.experimental.pallas import tpu as pltpu

COMPUTE_ONLY = os.environ.get("KERNEL_COMPUTE_ONLY") == "1"

N_DEV = 4
M_BLK = 2048
M_HALF = 1024
D = 2048
F_SHARD = 8192
F_CHUNK = 512
K_GROUP = 2048
N_FC = F_SHARD // F_CHUNK
K_PER_G = K_GROUP // F_CHUNK
N_G = F_SHARD // K_GROUP

BF16 = jnp.bfloat16
F32 = jnp.float32


def kernel(x, W1, W2):
    xb = x.astype(BF16)
    w1b = W1.astype(BF16)
    w2b = W2.astype(BF16)

    def body(x_hbm, w1_hbm, w2_hbm, out_hbm, ag_hbm, rs_hbm,
             xstage, w1buf, w2buf, hbuf, acc, rs_send,
             ag_send_sems, ag_recv_sems, rs_send_sems, rs_recv_sems,
             w1sems, w2sems, local_sem):
        me = lax.axis_index("i")

        if not COMPUTE_ONLY:
            barrier = pltpu.get_barrier_semaphore()
            for d in range(1, N_DEV):
                pl.semaphore_signal(
                    barrier, inc=1,
                    device_id=((me + d) % N_DEV,),
                    device_id_type=pl.DeviceIdType.MESH,
                )
            pl.semaphore_wait(barrier, N_DEV - 1)

        ag_sends = []
        for d in range(1, N_DEV) if not COMPUTE_ONLY else []:
            for half in range(2):
                rdma = pltpu.make_async_remote_copy(
                    src_ref=x_hbm.at[pl.ds(half * M_HALF, M_HALF), :],
                    dst_ref=ag_hbm.at[d - 1, pl.ds(half * M_HALF, M_HALF), :],
                    send_sem=ag_send_sems.at[d - 1, half],
                    recv_sem=ag_recv_sems.at[d - 1, half],
                    device_id=((me + d) % N_DEV,),
                    device_id_type=pl.DeviceIdType.MESH,
                )
                rdma.start()
                ag_sends.append(rdma)

        def start_w1(fc, slot):
            pltpu.make_async_copy(
                w1_hbm.at[:, pl.ds(fc * F_CHUNK, F_CHUNK)],
                w1buf.at[slot], w1sems.at[slot]).start()

        def wait_w1(slot):
            pltpu.make_async_copy(
                w1_hbm.at[:, pl.ds(0, F_CHUNK)],
                w1buf.at[slot], w1sems.at[slot]).wait()

        def start_w2(g, slot):
            pltpu.make_async_copy(
                w2_hbm.at[pl.ds(g * K_GROUP, K_GROUP), :],
                w2buf.at[slot], w2sems.at[slot]).start()

        def wait_w2(slot):
            pltpu.make_async_copy(
                w2_hbm.at[pl.ds(0, K_GROUP), :],
                w2buf.at[slot], w2sems.at[slot]).wait()

        def compute_half_into_acc(half):
            acc[...] = jnp.zeros_like(acc)

            def silu_store(h, k):
                s = jnp.tanh(h * 0.5)
                hbuf[:, pl.ds(k * F_CHUNK, F_CHUNK)] = (
                    (h * 0.5) * (s + 1.0)).astype(BF16)

            def g_body(g, carry):
                gcur = lax.rem(g, 2)
                xh = xstage[...]
                h_prev = None
                for k in range(K_PER_G):
                    fc = g * K_PER_G + k
                    kcur = lax.rem(fc, 2)
                    start_w1(lax.rem(fc + 1, N_FC), 1 - kcur)
                    wait_w1(kcur)
                    h = jnp.dot(xh, w1buf[kcur],
                                preferred_element_type=F32)
                    if h_prev is not None:
                        silu_store(h_prev, k - 1)
                    h_prev = h
                silu_store(h_prev, K_PER_G - 1)

                start_w2(lax.rem(g + 1, N_G), 1 - gcur)
                wait_w2(gcur)
                p = jnp.dot(hbuf[...], w2buf[gcur],
                            preferred_element_type=F32)
                acc[...] = acc[...] + p
                return carry

            lax.fori_loop(0, N_G, g_body, 0)

        schedule = [(0, 0), (1, 0), (1, 1), (2, 0), (2, 1),
                    (3, 0), (3, 1), (0, 1)]
        rs_inflight = [None]
        start_w1(0, 0)
        start_w2(0, 0)
        for d, half in schedule:
            if d != 0 and not COMPUTE_ONLY:
                slot = 3 - d
                recv = pltpu.make_async_remote_copy(
                    src_ref=x_hbm.at[pl.ds(half * M_HALF, M_HALF), :],
                    dst_ref=ag_hbm.at[slot, pl.ds(half * M_HALF, M_HALF), :],
                    send_sem=ag_send_sems.at[slot, half],
                    recv_sem=ag_recv_sems.at[slot, half],
                    device_id=(me,),
                    device_id_type=pl.DeviceIdType.MESH,
                )
                recv.wait_recv()
                xsrc = ag_hbm.at[slot]
            else:
                xsrc = x_hbm
            cp = pltpu.make_async_copy(
                xsrc.at[pl.ds(half * M_HALF, M_HALF), :], xstage,
                local_sem)
            cp.start()
            cp.wait()
            compute_half_into_acc(half)
            if d == 0 and half == 0:
                if rs_inflight[0] is not None:
                    rs_inflight[0].wait_send()
                    rs_inflight[0] = None
                rs_send[...] = acc[...].astype(BF16)
                cp = pltpu.make_async_copy(
                    rs_send,
                    rs_hbm.at[3, pl.ds(half * M_HALF, M_HALF), :],
                    local_sem)
                cp.start()
                cp.wait()
            elif d != 0 and not COMPUTE_ONLY:
                if rs_inflight[0] is not None:
                    rs_inflight[0].wait_send()
                rs_send[...] = acc[...].astype(BF16)
                rdma = pltpu.make_async_remote_copy(
                    src_ref=rs_send,
                    dst_ref=rs_hbm.at[d - 1,
                                      pl.ds(half * M_HALF, M_HALF), :],
                    send_sem=rs_send_sems.at[0],
                    recv_sem=rs_recv_sems.at[d - 1, half],
                    device_id=((me + d) % N_DEV,),
                    device_id_type=pl.DeviceIdType.MESH,
                )
                rdma.start()
                rs_inflight[0] = rdma

        if rs_inflight[0] is not None:
            rs_inflight[0].wait_send()

        for half in range(2):
            if half == 0:
                cp = pltpu.make_async_copy(
                    rs_hbm.at[3, pl.ds(half * M_HALF, M_HALF), :],
                    rs_send, local_sem)
                cp.start()
                cp.wait()
                accv = rs_send[...].astype(F32)
            else:
                accv = acc[...]
            for s in range(3) if not COMPUTE_ONLY else []:
                recv = pltpu.make_async_remote_copy(
                    src_ref=rs_send,
                    dst_ref=rs_hbm.at[s, pl.ds(half * M_HALF, M_HALF), :],
                    send_sem=rs_send_sems.at[0],
                    recv_sem=rs_recv_sems.at[s, half],
                    device_id=(me,),
                    device_id_type=pl.DeviceIdType.MESH,
                )
                recv.wait_recv()
                cp = pltpu.make_async_copy(
                    rs_hbm.at[s, pl.ds(half * M_HALF, M_HALF), :],
                    rs_send, local_sem)
                cp.start()
                cp.wait()
                accv = accv + rs_send[...].astype(F32)
            rs_send[...] = accv.astype(BF16)
            cp = pltpu.make_async_copy(
                rs_send,
                out_hbm.at[pl.ds(half * M_HALF, M_HALF), :],
                local_sem)
            cp.start()
            cp.wait()

        wait_w1(0)
        wait_w2(0)

        for rdma in ag_sends:
            rdma.wait_send()

    out, _, _ = pl.pallas_call(
        body,
        out_shape=(
            jax.ShapeDtypeStruct((M_BLK, D), BF16),
            jax.ShapeDtypeStruct((N_DEV - 1, M_BLK, D), BF16),
            jax.ShapeDtypeStruct((N_DEV, M_BLK, D), BF16),
        ),
        in_specs=[
            pl.BlockSpec(memory_space=pltpu.HBM),
            pl.BlockSpec(memory_space=pltpu.HBM),
            pl.BlockSpec(memory_space=pltpu.HBM),
        ],
        out_specs=(
            pl.BlockSpec(memory_space=pltpu.HBM),
            pl.BlockSpec(memory_space=pltpu.HBM),
            pl.BlockSpec(memory_space=pltpu.HBM),
        ),
        scratch_shapes=[
            pltpu.VMEM((M_HALF, D), BF16),
            pltpu.VMEM((2, D, F_CHUNK), BF16),
            pltpu.VMEM((2, K_GROUP, D), BF16),
            pltpu.VMEM((M_HALF, K_GROUP), BF16),
            pltpu.VMEM((M_HALF, D), F32),
            pltpu.VMEM((M_HALF, D), BF16),
            pltpu.SemaphoreType.DMA((N_DEV - 1, 2)),
            pltpu.SemaphoreType.DMA((N_DEV - 1, 2)),
            pltpu.SemaphoreType.DMA((2,)),
            pltpu.SemaphoreType.DMA((N_DEV - 1, 2)),
            pltpu.SemaphoreType.DMA((2,)),
            pltpu.SemaphoreType.DMA((2,)),
            pltpu.SemaphoreType.DMA,
        ],
        compiler_params=pltpu.CompilerParams(
            collective_id=None if COMPUTE_ONLY else 0,
            vmem_limit_bytes=62 * 1024 * 1024,
        ),
    )(xb, w1b, w2b)
    return out


# device time: 811962 ns/iter; 1.0673x vs baseline; 1.0673x over previous
import os

import jax
import jax.numpy as jnp
from jax import lax
from jax.experimental import pallas as pl
from jax.experimental.pallas import tpu as pltpu

COMPUTE_ONLY = os.environ.get("KERNEL_COMPUTE_ONLY") == "1"

N_DEV = 4
M_BLK = 2048
M_HALF = 1024
D = 2048
F_SHARD = 8192
F_CHUNK = 512
K_GROUP = 2048
N_FC = F_SHARD // F_CHUNK
K_PER_G = K_GROUP // F_CHUNK
N_G = F_SHARD // K_GROUP

BF16 = jnp.bfloat16
F32 = jnp.float32


def kernel(x, W1, W2):
    xb = x.astype(BF16)
    w1b = W1.astype(BF16)
    w2b = W2.astype(BF16)

    def body(x_hbm, w1_hbm, w2_hbm, out_hbm, ag_hbm, rs_hbm,
             xstage, w1buf, w2buf, hbuf, acc, rs_send,
             ag_send_sems, ag_recv_sems, rs_send_sems, rs_recv_sems,
             w1sems, w2sems, local_sem):
        me = lax.axis_index("i")

        if not COMPUTE_ONLY:
            barrier = pltpu.get_barrier_semaphore()
            for d in range(1, N_DEV):
                pl.semaphore_signal(
                    barrier, inc=1,
                    device_id=((me + d) % N_DEV,),
                    device_id_type=pl.DeviceIdType.MESH,
                )
            pl.semaphore_wait(barrier, N_DEV - 1)

        ag_sends = []
        for d in range(1, N_DEV) if not COMPUTE_ONLY else []:
            for half in range(2):
                rdma = pltpu.make_async_remote_copy(
                    src_ref=x_hbm.at[pl.ds(half * M_HALF, M_HALF), :],
                    dst_ref=ag_hbm.at[d - 1, pl.ds(half * M_HALF, M_HALF), :],
                    send_sem=ag_send_sems.at[d - 1, half],
                    recv_sem=ag_recv_sems.at[d - 1, half],
                    device_id=((me + d) % N_DEV,),
                    device_id_type=pl.DeviceIdType.MESH,
                )
                rdma.start()
                ag_sends.append(rdma)

        def start_w1(fc, slot):
            pltpu.make_async_copy(
                w1_hbm.at[:, pl.ds(fc * F_CHUNK, F_CHUNK)],
                w1buf.at[slot], w1sems.at[slot]).start()

        def wait_w1(slot):
            pltpu.make_async_copy(
                w1_hbm.at[:, pl.ds(0, F_CHUNK)],
                w1buf.at[slot], w1sems.at[slot]).wait()

        def start_w2(g, slot):
            pltpu.make_async_copy(
                w2_hbm.at[pl.ds(g * K_GROUP, K_GROUP), :],
                w2buf.at[slot], w2sems.at[slot]).start()

        def wait_w2(slot):
            pltpu.make_async_copy(
                w2_hbm.at[pl.ds(0, K_GROUP), :],
                w2buf.at[slot], w2sems.at[slot]).wait()

        def compute_half_into_acc(half):
            acc[...] = jnp.zeros_like(acc)

            def g_body(g, carry):
                gcur = lax.rem(g, 2)
                xh = xstage[...]
                for k in range(K_PER_G):
                    fc = g * K_PER_G + k
                    kcur = lax.rem(fc, 2)
                    start_w1(lax.rem(fc + 1, N_FC), 1 - kcur)
                    wait_w1(kcur)
                    h = jnp.dot(xh, w1buf[kcur],
                                preferred_element_type=F32)
                    s = jnp.tanh(h * 0.5)
                    hbuf[:, pl.ds(k * F_CHUNK, F_CHUNK)] = (
                        (h * 0.5) * (s + 1.0)).astype(BF16)

                start_w2(lax.rem(g + 1, N_G), 1 - gcur)
                wait_w2(gcur)
                p = jnp.dot(hbuf[...], w2buf[gcur],
                            preferred_element_type=F32)
                acc[...] = acc[...] + p
                return carry

            lax.fori_loop(0, N_G, g_body, 0)

        schedule = [(0, 0), (1, 0), (1, 1), (2, 0), (2, 1),
                    (3, 0), (3, 1), (0, 1)]
        rs_inflight = [None, None]
        start_w1(0, 0)
        start_w2(0, 0)
        for d, half in schedule:
            if d != 0 and not COMPUTE_ONLY:
                slot = 3 - d
                recv = pltpu.make_async_remote_copy(
                    src_ref=x_hbm.at[pl.ds(half * M_HALF, M_HALF), :],
                    dst_ref=ag_hbm.at[slot, pl.ds(half * M_HALF, M_HALF), :],
                    send_sem=ag_send_sems.at[slot, half],
                    recv_sem=ag_recv_sems.at[slot, half],
                    device_id=(me,),
                    device_id_type=pl.DeviceIdType.MESH,
                )
                recv.wait_recv()
                xsrc = ag_hbm.at[slot]
            else:
                xsrc = x_hbm
            cp = pltpu.make_async_copy(
                xsrc.at[pl.ds(half * M_HALF, M_HALF), :], xstage,
                local_sem)
            cp.start()
            cp.wait()
            compute_half_into_acc(half)
            if d == 0 and half == 0:
                rs_send[half] = acc[...].astype(BF16)
                cp = pltpu.make_async_copy(
                    rs_send.at[half],
                    rs_hbm.at[3, pl.ds(half * M_HALF, M_HALF), :],
                    local_sem)
                cp.start()
                cp.wait()
            elif d != 0 and not COMPUTE_ONLY:
                if rs_inflight[half] is not None:
                    rs_inflight[half].wait_send()
                rs_send[half] = acc[...].astype(BF16)
                rdma = pltpu.make_async_remote_copy(
                    src_ref=rs_send.at[half],
                    dst_ref=rs_hbm.at[d - 1,
                                      pl.ds(half * M_HALF, M_HALF), :],
                    send_sem=rs_send_sems.at[half],
                    recv_sem=rs_recv_sems.at[d - 1, half],
                    device_id=((me + d) % N_DEV,),
                    device_id_type=pl.DeviceIdType.MESH,
                )
                rdma.start()
                rs_inflight[half] = rdma

        for half in range(2):
            if rs_inflight[half] is not None:
                rs_inflight[half].wait_send()

        for half in range(2):
            if half == 0:
                cp = pltpu.make_async_copy(
                    rs_hbm.at[3, pl.ds(half * M_HALF, M_HALF), :],
                    rs_send.at[half], local_sem)
                cp.start()
                cp.wait()
                accv = rs_send[half].astype(F32)
            else:
                accv = acc[...]
            for s in range(3) if not COMPUTE_ONLY else []:
                recv = pltpu.make_async_remote_copy(
                    src_ref=rs_send.at[half],
                    dst_ref=rs_hbm.at[s, pl.ds(half * M_HALF, M_HALF), :],
                    send_sem=rs_send_sems.at[half],
                    recv_sem=rs_recv_sems.at[s, half],
                    device_id=(me,),
                    device_id_type=pl.DeviceIdType.MESH,
                )
                recv.wait_recv()
                cp = pltpu.make_async_copy(
                    rs_hbm.at[s, pl.ds(half * M_HALF, M_HALF), :],
                    rs_send.at[half], local_sem)
                cp.start()
                cp.wait()
                accv = accv + rs_send[half].astype(F32)
            rs_send[half] = accv.astype(BF16)
            cp = pltpu.make_async_copy(
                rs_send.at[half],
                out_hbm.at[pl.ds(half * M_HALF, M_HALF), :],
                local_sem)
            cp.start()
            cp.wait()

        wait_w1(0)
        wait_w2(0)

        for rdma in ag_sends:
            rdma.wait_send()

    out, _, _ = pl.pallas_call(
        body,
        out_shape=(
            jax.ShapeDtypeStruct((M_BLK, D), BF16),
            jax.ShapeDtypeStruct((N_DEV - 1, M_BLK, D), BF16),
            jax.ShapeDtypeStruct((N_DEV, M_BLK, D), BF16),
        ),
        in_specs=[
            pl.BlockSpec(memory_space=pltpu.HBM),
            pl.BlockSpec(memory_space=pltpu.HBM),
            pl.BlockSpec(memory_space=pltpu.HBM),
        ],
        out_specs=(
            pl.BlockSpec(memory_space=pltpu.HBM),
            pl.BlockSpec(memory_space=pltpu.HBM),
            pl.BlockSpec(memory_space=pltpu.HBM),
        ),
        scratch_shapes=[
            pltpu.VMEM((M_HALF, D), BF16),
            pltpu.VMEM((2, D, F_CHUNK), BF16),
            pltpu.VMEM((2, K_GROUP, D), BF16),
            pltpu.VMEM((M_HALF, K_GROUP), BF16),
            pltpu.VMEM((M_HALF, D), F32),
            pltpu.VMEM((2, M_HALF, D), BF16),
            pltpu.SemaphoreType.DMA((N_DEV - 1, 2)),
            pltpu.SemaphoreType.DMA((N_DEV - 1, 2)),
            pltpu.SemaphoreType.DMA((2,)),
            pltpu.SemaphoreType.DMA((N_DEV - 1, 2)),
            pltpu.SemaphoreType.DMA((2,)),
            pltpu.SemaphoreType.DMA((2,)),
            pltpu.SemaphoreType.DMA,
        ],
        compiler_params=pltpu.CompilerParams(
            collective_id=None if COMPUTE_ONLY else 0,
            vmem_limit_bytes=62 * 1024 * 1024,
        ),
    )(xb, w1b, w2b)
    return out
